# baseline (speedup 1.0000x reference)
import functools

import jax
import jax.numpy as jnp
from jax import lax
from jax.experimental import pallas as pl
from jax.experimental.pallas import tpu as pltpu
from jax.experimental.pallas import tpu_sc as plsc


def kernel(frames):
    B, C, T, H, W = frames.shape
    S = T // 4
    ROWS = B * C * S
    NW = 32
    RPW = ROWS // NW
    D = H * W

    CPR = 4
    CH = D // CPR
    TOT = RPW * CPR
    NBUF = 6
    AHEAD = 3
    LAG = NBUF - AHEAD

    flat = frames.reshape(B * C * T, CPR, CH)
    mesh = plsc.VectorSubcoreMesh(core_axis_name="c", subcore_axis_name="s")

    @functools.partial(
        pl.kernel,
        out_type=jax.ShapeDtypeStruct((ROWS, CPR, CH), frames.dtype),
        mesh=mesh,
        scratch_types=[
            pltpu.VMEM((NBUF, CH), frames.dtype),
            pltpu.SemaphoreType.DMA((NBUF,)),
            pltpu.SemaphoreType.DMA((NBUF,)),
        ],
    )
    def pack_slow(src_hbm, out_hbm, buf, sin, sout):
        wid = lax.axis_index("s") * 2 + lax.axis_index("c")
        base = wid * RPW

        def gather(j):
            i, c = j // CPR, j % CPR
            r = base + i
            bc = r // S
            tp = r % S
            src_row = bc * T + (tp * (T - 1)) // (S - 1)
            return pltpu.make_async_copy(src_hbm.at[src_row, c],
                                         buf.at[j % NBUF], sin.at[j % NBUF])

        def scatter(j):
            i, c = j // CPR, j % CPR
            return pltpu.make_async_copy(buf.at[j % NBUF],
                                         out_hbm.at[base + i, c],
                                         sout.at[j % NBUF])

        waited = set()
        for j in range(min(AHEAD, TOT)):
            gather(j).start()
        for j in range(TOT):
            gather(j).wait()
            scatter(j).start()
            k = j + AHEAD
            if k < TOT:
                p = k - NBUF
                if p >= 0:
                    scatter(p).wait()
                    waited.add(p)
                gather(k).start()
        for j in range(TOT):
            if j not in waited:
                scatter(j).wait()

    slow = pack_slow(flat).reshape(B, C, S, H, W)
    return (slow, frames)


# speedup vs baseline: 1.1539x; 1.1539x over previous
import functools

import jax
import jax.numpy as jnp
from jax import lax
from jax.experimental import pallas as pl
from jax.experimental.pallas import tpu as pltpu
from jax.experimental.pallas import tpu_sc as plsc


def kernel(frames):
    B, C, T, H, W = frames.shape
    S = T // 4
    ROWS = B * C * S
    NW = 32
    RPW = ROWS // NW
    D = H * W

    NS = 16
    NBUF = 2

    flat = frames.reshape(B * C * T, D)
    mesh = plsc.VectorSubcoreMesh(core_axis_name="c", subcore_axis_name="s")

    @functools.partial(
        pl.kernel,
        out_type=jax.ShapeDtypeStruct((ROWS, D), frames.dtype),
        mesh=mesh,
        scratch_types=[
            pltpu.VMEM_SHARED((NS, NBUF, D), frames.dtype),
            pltpu.SemaphoreType.DMA((NBUF,)),
            pltpu.SemaphoreType.DMA((NBUF,)),
        ],
    )
    def pack_slow(src_hbm, out_hbm, buf, sin, sout):
        cid = lax.axis_index("c")
        sid = lax.axis_index("s")
        wid = sid * 2 + cid
        base = wid * RPW

        def gather(i):
            r = base + i
            bc = r // S
            tp = r % S
            src_row = bc * T + (tp * (T - 1)) // (S - 1)
            return pltpu.make_async_copy(src_hbm.at[src_row],
                                         buf.at[sid, i % NBUF],
                                         sin.at[i % NBUF])

        def scatter(i):
            return pltpu.make_async_copy(buf.at[sid, i % NBUF],
                                         out_hbm.at[base + i],
                                         sout.at[i % NBUF])

        gather(0).start()
        for i in range(RPW):
            if i + 1 < RPW:
                if i >= 1:
                    scatter(i - 1).wait()
                gather(i + 1).start()
            gather(i).wait()
            scatter(i).start()
        scatter(RPW - 2).wait()
        scatter(RPW - 1).wait()

    slow = pack_slow(flat).reshape(B, C, S, H, W)
    return (slow, frames)
